# baseline (speedup 1.0000x reference)
import functools

import jax
import jax.numpy as jnp
from jax import lax
from jax.experimental import pallas as pl
from jax.experimental.pallas import tpu as pltpu
from jax.experimental.pallas import tpu_sc as plsc

_B = 16384
_ROBERTA_DIM = 768
_EMB_DIM = 4
_D_IN = _ROBERTA_DIM + 1 + _EMB_DIM
_D_H = _D_IN // 2

_NC = 2
_NS = 16
_NW = _NC * _NS
_CHUNK = 128
_ROWS_PER_W = _B // _NW
_CHUNKS_PER_W = _ROWS_PER_W // _CHUNK


_ELEMS_PER_W = _EMB_DIM * _B // _NW
_ECHUNKS_PER_W = _ELEMS_PER_W // _CHUNK


def _sc_gather(fidx_hbm, tabf_hbm, out_hbm, fidx_v, vals_v, sem):
    wid = lax.axis_index("s") * _NC + lax.axis_index("c")
    pltpu.sync_copy(fidx_hbm.at[pl.ds(wid * _ECHUNKS_PER_W, _ECHUNKS_PER_W)],
                    fidx_v)
    copies = []
    for j in range(_ECHUNKS_PER_W):
        copies.append(
            pltpu.async_copy(
                tabf_hbm.at[fidx_v.at[j]],
                vals_v.at[pl.ds(j * _CHUNK, _CHUNK)],
                sem,
            )
        )
    for c in copies:
        c.wait()
    pltpu.sync_copy(vals_v, out_hbm.at[pl.ds(wid * _ELEMS_PER_W,
                                             _ELEMS_PER_W)])


@jax.jit
def _gather_rows(user_indices, user_emb_table):
    fidx = (jnp.arange(4, dtype=jnp.int32)[:, None] * 1000000
            + user_indices[None, :])
    fidx = fidx.reshape(_EMB_DIM * _B // _CHUNK, _CHUNK)
    tabf = user_emb_table.T.reshape(-1)
    mesh = plsc.VectorSubcoreMesh(core_axis_name="c", subcore_axis_name="s")
    k = pl.kernel(
        _sc_gather,
        out_type=jax.ShapeDtypeStruct((_B * _EMB_DIM,), jnp.float32),
        mesh=mesh,
        scratch_types=[
            pltpu.VMEM((_ECHUNKS_PER_W, _CHUNK), jnp.int32),
            pltpu.VMEM((_ELEMS_PER_W,), jnp.float32),
            pltpu.SemaphoreType.DMA,
        ],
    )
    return k(fidx, tabf).reshape(_EMB_DIM, _B)


def _mlp_body(cls_ref, usr_ref, isw_ref, w1a_ref, w1b_ref, w1c_ref, b1_ref,
              w2_ref, b2_ref, aro_ref, val_ref):
    x = cls_ref[...]
    acc = jnp.dot(x, w1a_ref[...], preferred_element_type=jnp.float32)
    acc += lax.dot_general(usr_ref[...], w1b_ref[...],
                           (((0,), (0,)), ((), ())),
                           preferred_element_type=jnp.float32)
    acc += isw_ref[...] * w1c_ref[...]
    acc += b1_ref[...]
    h = 0.5 * acc * (1.0 + lax.erf(acc * 0.7071067811865476))
    logits = jnp.dot(h, w2_ref[...], preferred_element_type=jnp.float32)
    logits += b2_ref[...]
    probs = jax.nn.sigmoid(logits)
    aro_ref[...] = probs[:, 1]
    val_ref[...] = probs[:, 0]


@jax.jit
def _mlp(cls_embeddings, user_matrix_t, is_word_indices, W1, b1, W2, b2):
    bb = 4096
    grid = (_B // bb,)
    w1a = W1[:_ROBERTA_DIM]
    w1b = W1[_ROBERTA_DIM:_ROBERTA_DIM + _EMB_DIM]
    w1c = W1[_ROBERTA_DIM + _EMB_DIM:]
    return pl.pallas_call(
        _mlp_body,
        grid=grid,
        in_specs=[
            pl.BlockSpec((bb, _ROBERTA_DIM), lambda i: (i, 0)),
            pl.BlockSpec((_EMB_DIM, bb), lambda i: (0, i)),
            pl.BlockSpec((bb, 1), lambda i: (i, 0)),
            pl.BlockSpec((_ROBERTA_DIM, _D_H), lambda i: (0, 0)),
            pl.BlockSpec((_EMB_DIM, _D_H), lambda i: (0, 0)),
            pl.BlockSpec((1, _D_H), lambda i: (0, 0)),
            pl.BlockSpec((1, _D_H), lambda i: (0, 0)),
            pl.BlockSpec((_D_H, 2), lambda i: (0, 0)),
            pl.BlockSpec((1, 2), lambda i: (0, 0)),
        ],
        out_specs=[
            pl.BlockSpec((bb,), lambda i: (i,)),
            pl.BlockSpec((bb,), lambda i: (i,)),
        ],
        out_shape=[
            jax.ShapeDtypeStruct((_B,), jnp.float32),
            jax.ShapeDtypeStruct((_B,), jnp.float32),
        ],
    )(cls_embeddings, user_matrix_t, is_word_indices, w1a, w1b, w1c,
      b1.reshape(1, _D_H), W2, b2.reshape(1, 2))


def kernel(cls_embeddings, user_indices, is_word_indices, user_emb_table,
           W1, b1, W2, b2):
    user_matrix_t = _gather_rows(user_indices, user_emb_table)
    arousal, valence = _mlp(cls_embeddings, user_matrix_t, is_word_indices,
                            W1, b1, W2, b2)
    return (arousal, valence)


# speedup vs baseline: 1.0039x; 1.0039x over previous
import functools

import jax
import jax.numpy as jnp
from jax import lax
from jax.experimental import pallas as pl
from jax.experimental.pallas import tpu as pltpu
from jax.experimental.pallas import tpu_sc as plsc

_B = 16384
_ROBERTA_DIM = 768
_EMB_DIM = 4
_D_IN = _ROBERTA_DIM + 1 + _EMB_DIM
_D_H = _D_IN // 2

_NC = 2
_NS = 16
_NW = _NC * _NS
_CHUNK = 128
_ROWS_PER_W = _B // _NW
_CHUNKS_PER_W = _ROWS_PER_W // _CHUNK


_ELEMS_PER_W = _EMB_DIM * _B // _NW
_ECHUNKS_PER_W = _ELEMS_PER_W // _CHUNK


def _sc_gather(fidx_hbm, tabf_hbm, out_hbm, fidx_v, vals_v, sem):
    wid = lax.axis_index("s") * _NC + lax.axis_index("c")
    pltpu.sync_copy(fidx_hbm.at[pl.ds(wid * _ECHUNKS_PER_W, _ECHUNKS_PER_W)],
                    fidx_v)
    copies = []
    for j in range(_ECHUNKS_PER_W):
        copies.append(
            pltpu.async_copy(
                tabf_hbm.at[fidx_v.at[j]],
                vals_v.at[pl.ds(j * _CHUNK, _CHUNK)],
                sem,
            )
        )
    for c in copies:
        c.wait()
    pltpu.sync_copy(vals_v, out_hbm.at[pl.ds(wid * _ELEMS_PER_W,
                                             _ELEMS_PER_W)])


@jax.jit
def _gather_rows(user_indices, user_emb_table):
    fidx = (jnp.arange(4, dtype=jnp.int32)[:, None] * 1000000
            + user_indices[None, :])
    fidx = fidx.reshape(_EMB_DIM * _B // _CHUNK, _CHUNK)
    tabf = user_emb_table.T.reshape(-1)
    mesh = plsc.VectorSubcoreMesh(core_axis_name="c", subcore_axis_name="s")
    k = pl.kernel(
        _sc_gather,
        out_type=jax.ShapeDtypeStruct((_B * _EMB_DIM,), jnp.float32),
        mesh=mesh,
        scratch_types=[
            pltpu.VMEM((_ECHUNKS_PER_W, _CHUNK), jnp.int32),
            pltpu.VMEM((_ELEMS_PER_W,), jnp.float32),
            pltpu.SemaphoreType.DMA,
        ],
    )
    return k(fidx, tabf).reshape(_EMB_DIM, _B)


def _mlp_body(cls_ref, usr_ref, isw_ref, w1a_ref, w1b_ref, w1c_ref, b1_ref,
              w2_ref, b2_ref, aro_ref, val_ref):
    x = cls_ref[...]
    acc = jnp.dot(x, w1a_ref[...], preferred_element_type=jnp.float32)
    acc += lax.dot_general(usr_ref[...], w1b_ref[...],
                           (((0,), (0,)), ((), ())),
                           preferred_element_type=jnp.float32)
    acc += isw_ref[...] * w1c_ref[...]
    acc += b1_ref[...]
    h = 0.5 * acc * (1.0 + lax.erf(acc * 0.7071067811865476))
    logits = jnp.dot(h, w2_ref[...], preferred_element_type=jnp.float32)
    logits += b2_ref[...]
    probs = jax.nn.sigmoid(logits)
    aro_ref[...] = probs[:, 1]
    val_ref[...] = probs[:, 0]


@jax.jit
def _mlp(cls_embeddings, user_matrix_t, is_word_indices, W1, b1, W2, b2):
    bb = 1024
    grid = (_B // bb,)
    w1a = W1[:_ROBERTA_DIM]
    w1b = W1[_ROBERTA_DIM:_ROBERTA_DIM + _EMB_DIM]
    w1c = W1[_ROBERTA_DIM + _EMB_DIM:]
    return pl.pallas_call(
        _mlp_body,
        grid=grid,
        in_specs=[
            pl.BlockSpec((bb, _ROBERTA_DIM), lambda i: (i, 0)),
            pl.BlockSpec((_EMB_DIM, bb), lambda i: (0, i)),
            pl.BlockSpec((bb, 1), lambda i: (i, 0)),
            pl.BlockSpec((_ROBERTA_DIM, _D_H), lambda i: (0, 0)),
            pl.BlockSpec((_EMB_DIM, _D_H), lambda i: (0, 0)),
            pl.BlockSpec((1, _D_H), lambda i: (0, 0)),
            pl.BlockSpec((1, _D_H), lambda i: (0, 0)),
            pl.BlockSpec((_D_H, 2), lambda i: (0, 0)),
            pl.BlockSpec((1, 2), lambda i: (0, 0)),
        ],
        out_specs=[
            pl.BlockSpec((bb,), lambda i: (i,)),
            pl.BlockSpec((bb,), lambda i: (i,)),
        ],
        out_shape=[
            jax.ShapeDtypeStruct((_B,), jnp.float32),
            jax.ShapeDtypeStruct((_B,), jnp.float32),
        ],
    )(cls_embeddings, user_matrix_t, is_word_indices, w1a, w1b, w1c,
      b1.reshape(1, _D_H), W2, b2.reshape(1, 2))


def kernel(cls_embeddings, user_indices, is_word_indices, user_emb_table,
           W1, b1, W2, b2):
    user_matrix_t = _gather_rows(user_indices, user_emb_table)
    arousal, valence = _mlp(cls_embeddings, user_matrix_t, is_word_indices,
                            W1, b1, W2, b2)
    return (arousal, valence)


# speedup vs baseline: 1.0752x; 1.0710x over previous
import functools

import jax
import jax.numpy as jnp
from jax import lax
from jax.experimental import pallas as pl
from jax.experimental.pallas import tpu as pltpu
from jax.experimental.pallas import tpu_sc as plsc

_B = 16384
_ROBERTA_DIM = 768
_EMB_DIM = 4
_D_IN = _ROBERTA_DIM + 1 + _EMB_DIM
_D_H = _D_IN // 2

_NC = 2
_NS = 16
_NW = _NC * _NS
_CHUNK = 128
_ROWS_PER_W = _B // _NW
_CHUNKS_PER_W = _ROWS_PER_W // _CHUNK


_ELEMS_PER_W = _EMB_DIM * _B // _NW
_ECHUNKS_PER_W = _ELEMS_PER_W // _CHUNK


def _sc_gather(fidx_hbm, tabf_hbm, out_hbm, fidx_v, vals_v, sem):
    wid = lax.axis_index("s") * _NC + lax.axis_index("c")
    pltpu.sync_copy(fidx_hbm.at[pl.ds(wid * _ECHUNKS_PER_W, _ECHUNKS_PER_W)],
                    fidx_v)
    copies = []
    for j in range(_ECHUNKS_PER_W):
        copies.append(
            pltpu.async_copy(
                tabf_hbm.at[fidx_v.at[j]],
                vals_v.at[pl.ds(j * _CHUNK, _CHUNK)],
                sem,
            )
        )
    for c in copies:
        c.wait()
    pltpu.sync_copy(vals_v, out_hbm.at[pl.ds(wid * _ELEMS_PER_W,
                                             _ELEMS_PER_W)])


@jax.jit
def _gather_rows(user_indices, user_emb_table):
    fidx = (jnp.arange(4, dtype=jnp.int32)[:, None] * 1000000
            + user_indices[None, :])
    fidx = fidx.reshape(_EMB_DIM * _B // _CHUNK, _CHUNK)
    tabf = user_emb_table.T.reshape(-1)
    mesh = plsc.VectorSubcoreMesh(core_axis_name="c", subcore_axis_name="s")
    k = pl.kernel(
        _sc_gather,
        out_type=jax.ShapeDtypeStruct((_B * _EMB_DIM,), jnp.float32),
        mesh=mesh,
        scratch_types=[
            pltpu.VMEM((_ECHUNKS_PER_W, _CHUNK), jnp.int32),
            pltpu.VMEM((_ELEMS_PER_W,), jnp.float32),
            pltpu.SemaphoreType.DMA,
        ],
    )
    return k(fidx, tabf).reshape(_EMB_DIM, _B)


def _mlp_body(cls_ref, usr_ref, isw_ref, w1a_ref, w1b_ref, w1c_ref, b1_ref,
              w2_ref, b2_ref, aro_ref, val_ref):
    x = cls_ref[...]
    acc = jnp.dot(x, w1a_ref[...], preferred_element_type=jnp.float32)
    acc += jnp.dot(jnp.transpose(usr_ref[...]), w1b_ref[...],
                   preferred_element_type=jnp.float32)
    acc += isw_ref[...] * w1c_ref[...]
    acc += b1_ref[...]
    h = 0.5 * acc * (1.0 + lax.erf(acc * 0.7071067811865476))
    logits = jnp.dot(h, w2_ref[...], preferred_element_type=jnp.float32)
    logits_t = jnp.transpose(logits) + b2_ref[...]
    aro_ref[...] = jax.nn.sigmoid(logits_t[1])
    val_ref[...] = jax.nn.sigmoid(logits_t[0])


@jax.jit
def _mlp(cls_embeddings, user_matrix_t, is_word_indices, W1, b1, W2, b2):
    bb = 2048
    grid = (_B // bb,)
    w1a = W1[:_ROBERTA_DIM]
    w1b = W1[_ROBERTA_DIM:_ROBERTA_DIM + _EMB_DIM]
    w1c = W1[_ROBERTA_DIM + _EMB_DIM:]
    return pl.pallas_call(
        _mlp_body,
        grid=grid,
        in_specs=[
            pl.BlockSpec((bb, _ROBERTA_DIM), lambda i: (i, 0)),
            pl.BlockSpec((_EMB_DIM, bb), lambda i: (0, i)),
            pl.BlockSpec((bb, 1), lambda i: (i, 0)),
            pl.BlockSpec((_ROBERTA_DIM, _D_H), lambda i: (0, 0)),
            pl.BlockSpec((_EMB_DIM, _D_H), lambda i: (0, 0)),
            pl.BlockSpec((1, _D_H), lambda i: (0, 0)),
            pl.BlockSpec((1, _D_H), lambda i: (0, 0)),
            pl.BlockSpec((_D_H, 2), lambda i: (0, 0)),
            pl.BlockSpec((2, 1), lambda i: (0, 0)),
        ],
        out_specs=[
            pl.BlockSpec((bb,), lambda i: (i,)),
            pl.BlockSpec((bb,), lambda i: (i,)),
        ],
        out_shape=[
            jax.ShapeDtypeStruct((_B,), jnp.float32),
            jax.ShapeDtypeStruct((_B,), jnp.float32),
        ],
        compiler_params=pltpu.CompilerParams(
            fuse_transposed_lhs_in_matmul=True),
    )(cls_embeddings, user_matrix_t, is_word_indices, w1a, w1b, w1c,
      b1.reshape(1, _D_H), W2, b2.reshape(2, 1))


def kernel(cls_embeddings, user_indices, is_word_indices, user_emb_table,
           W1, b1, W2, b2):
    user_matrix_t = _gather_rows(user_indices, user_emb_table)
    arousal, valence = _mlp(cls_embeddings, user_matrix_t, is_word_indices,
                            W1, b1, W2, b2)
    return (arousal, valence)


# speedup vs baseline: 1.1080x; 1.0305x over previous
import functools

import jax
import jax.numpy as jnp
from jax import lax
from jax.experimental import pallas as pl
from jax.experimental.pallas import tpu as pltpu
from jax.experimental.pallas import tpu_sc as plsc

_B = 16384
_ROBERTA_DIM = 768
_EMB_DIM = 4
_D_IN = _ROBERTA_DIM + 1 + _EMB_DIM
_D_H = _D_IN // 2

_NC = 2
_NS = 16
_NW = _NC * _NS
_CHUNK = 128
_ROWS_PER_W = _B // _NW
_CHUNKS_PER_W = _ROWS_PER_W // _CHUNK


_ELEMS_PER_W = _EMB_DIM * _B // _NW
_ECHUNKS_PER_W = _ELEMS_PER_W // _CHUNK


def _sc_gather(fidx_hbm, tabf_hbm, out_hbm, fidx_v, vals_v, sem):
    wid = lax.axis_index("s") * _NC + lax.axis_index("c")
    pltpu.sync_copy(fidx_hbm.at[pl.ds(wid * _ECHUNKS_PER_W, _ECHUNKS_PER_W)],
                    fidx_v)
    copies = []
    for j in range(_ECHUNKS_PER_W):
        copies.append(
            pltpu.async_copy(
                tabf_hbm.at[fidx_v.at[j]],
                vals_v.at[pl.ds(j * _CHUNK, _CHUNK)],
                sem,
            )
        )
    for c in copies:
        c.wait()
    pltpu.sync_copy(vals_v, out_hbm.at[pl.ds(wid * _ELEMS_PER_W,
                                             _ELEMS_PER_W)])


@jax.jit
def _gather_rows(user_indices, user_emb_table):
    fidx = (jnp.arange(4, dtype=jnp.int32)[:, None] * 1000000
            + user_indices[None, :])
    fidx = fidx.reshape(_EMB_DIM * _B // _CHUNK, _CHUNK)
    tabf = user_emb_table.T.reshape(-1)
    mesh = plsc.VectorSubcoreMesh(core_axis_name="c", subcore_axis_name="s")
    k = pl.kernel(
        _sc_gather,
        out_type=jax.ShapeDtypeStruct((_B * _EMB_DIM,), jnp.float32),
        mesh=mesh,
        scratch_types=[
            pltpu.VMEM((_ECHUNKS_PER_W, _CHUNK), jnp.int32),
            pltpu.VMEM((_ELEMS_PER_W,), jnp.float32),
            pltpu.SemaphoreType.DMA,
        ],
    )
    return k(fidx, tabf).reshape(_EMB_DIM, _B)


def _mlp_body(cls_ref, usr_ref, isw_ref, w1t_ref, b1_ref,
              w2_ref, b2_ref, aro_ref, val_ref):
    x = cls_ref[...]
    w1t = w1t_ref[...]
    acc = lax.dot_general(x, w1t[:, :_ROBERTA_DIM],
                          (((1,), (1,)), ((), ())),
                          preferred_element_type=jnp.float32)
    acc += lax.dot_general(
        jnp.transpose(usr_ref[...]),
        w1t[:, _ROBERTA_DIM:_ROBERTA_DIM + _EMB_DIM],
        (((1,), (1,)), ((), ())),
        preferred_element_type=jnp.float32)
    acc += isw_ref[...] * jnp.transpose(w1t[:, _D_IN - 1:])
    acc += b1_ref[...]
    h = 0.5 * acc * (1.0 + lax.erf(acc * 0.7071067811865476))
    logits = jnp.dot(h, w2_ref[...], preferred_element_type=jnp.float32)
    logits_t = jnp.transpose(logits) + b2_ref[...]
    aro_ref[...] = jax.nn.sigmoid(logits_t[1])
    val_ref[...] = jax.nn.sigmoid(logits_t[0])


@jax.jit
def _mlp(cls_embeddings, user_matrix_t, is_word_indices, W1, b1, W2, b2):
    bb = 2048
    grid = (_B // bb,)
    return pl.pallas_call(
        _mlp_body,
        grid=grid,
        in_specs=[
            pl.BlockSpec((bb, _ROBERTA_DIM), lambda i: (i, 0)),
            pl.BlockSpec((_EMB_DIM, bb), lambda i: (0, i)),
            pl.BlockSpec((bb, 1), lambda i: (i, 0)),
            pl.BlockSpec((_D_H, _D_IN), lambda i: (0, 0)),
            pl.BlockSpec((1, _D_H), lambda i: (0, 0)),
            pl.BlockSpec((_D_H, 2), lambda i: (0, 0)),
            pl.BlockSpec((2, 1), lambda i: (0, 0)),
        ],
        out_specs=[
            pl.BlockSpec((bb,), lambda i: (i,)),
            pl.BlockSpec((bb,), lambda i: (i,)),
        ],
        out_shape=[
            jax.ShapeDtypeStruct((_B,), jnp.float32),
            jax.ShapeDtypeStruct((_B,), jnp.float32),
        ],
        compiler_params=pltpu.CompilerParams(
            fuse_transposed_lhs_in_matmul=True),
    )(cls_embeddings, user_matrix_t, is_word_indices, W1.T,
      b1.reshape(1, _D_H), W2, b2.reshape(2, 1))


def kernel(cls_embeddings, user_indices, is_word_indices, user_emb_table,
           W1, b1, W2, b2):
    user_matrix_t = _gather_rows(user_indices, user_emb_table)
    arousal, valence = _mlp(cls_embeddings, user_matrix_t, is_word_indices,
                            W1, b1, W2, b2)
    return (arousal, valence)


# speedup vs baseline: 1.1100x; 1.0018x over previous
import functools

import jax
import jax.numpy as jnp
from jax import lax
from jax.experimental import pallas as pl
from jax.experimental.pallas import tpu as pltpu
from jax.experimental.pallas import tpu_sc as plsc

_B = 16384
_ROBERTA_DIM = 768
_EMB_DIM = 4
_D_IN = _ROBERTA_DIM + 1 + _EMB_DIM
_D_H = _D_IN // 2

_NC = 2
_NS = 16
_NW = _NC * _NS
_CHUNK = 128
_ROWS_PER_W = _B // _NW
_CHUNKS_PER_W = _ROWS_PER_W // _CHUNK


_ELEMS_PER_W = _EMB_DIM * _B // _NW
_ECHUNKS_PER_W = _ELEMS_PER_W // _CHUNK


def _sc_gather(fidx_hbm, tabf_hbm, out_hbm, fidx_v, vals_v, sem):
    wid = lax.axis_index("s") * _NC + lax.axis_index("c")
    pltpu.sync_copy(fidx_hbm.at[pl.ds(wid * _ECHUNKS_PER_W, _ECHUNKS_PER_W)],
                    fidx_v)
    copies = []
    for j in range(_ECHUNKS_PER_W):
        copies.append(
            pltpu.async_copy(
                tabf_hbm.at[fidx_v.at[j]],
                vals_v.at[pl.ds(j * _CHUNK, _CHUNK)],
                sem,
            )
        )
    for c in copies:
        c.wait()
    pltpu.sync_copy(vals_v, out_hbm.at[pl.ds(wid * _ELEMS_PER_W,
                                             _ELEMS_PER_W)])


@jax.jit
def _gather_rows(user_indices, user_emb_table):
    fidx = (jnp.arange(4, dtype=jnp.int32)[:, None] * 1000000
            + user_indices[None, :])
    fidx = fidx.reshape(_EMB_DIM * _B // _CHUNK, _CHUNK)
    tabf = user_emb_table.T.reshape(-1)
    mesh = plsc.VectorSubcoreMesh(core_axis_name="c", subcore_axis_name="s")
    k = pl.kernel(
        _sc_gather,
        out_type=jax.ShapeDtypeStruct((_B * _EMB_DIM,), jnp.float32),
        mesh=mesh,
        scratch_types=[
            pltpu.VMEM((_ECHUNKS_PER_W, _CHUNK), jnp.int32),
            pltpu.VMEM((_ELEMS_PER_W,), jnp.float32),
            pltpu.SemaphoreType.DMA,
        ],
    )
    return k(fidx, tabf).reshape(_EMB_DIM, _B)


def _mlp_body(cls_ref, usr_ref, isw_ref, w1t_ref, b1_ref,
              w2_ref, b2_ref, aro_ref, val_ref):
    x = cls_ref[...]
    w1t = w1t_ref[...]
    acc = lax.dot_general(x, w1t[:, :_ROBERTA_DIM],
                          (((1,), (1,)), ((), ())),
                          preferred_element_type=jnp.float32)
    acc += lax.dot_general(
        jnp.transpose(usr_ref[...]),
        w1t[:, _ROBERTA_DIM:_ROBERTA_DIM + _EMB_DIM],
        (((1,), (1,)), ((), ())),
        preferred_element_type=jnp.float32)
    acc += isw_ref[...] * jnp.transpose(w1t[:, _D_IN - 1:])
    acc += b1_ref[...]
    h = 0.5 * acc * (1.0 + lax.erf(acc * 0.7071067811865476))
    logits = jnp.dot(h, w2_ref[...], preferred_element_type=jnp.float32)
    logits_t = jnp.transpose(logits) + b2_ref[...]
    aro_ref[...] = jax.nn.sigmoid(logits_t[1])
    val_ref[...] = jax.nn.sigmoid(logits_t[0])


@jax.jit
def _mlp(cls_embeddings, user_matrix_t, is_word_indices, W1, b1, W2, b2):
    bb = 2048
    grid = (_B // bb,)
    return pl.pallas_call(
        _mlp_body,
        grid=grid,
        in_specs=[
            pl.BlockSpec((bb, _ROBERTA_DIM), lambda i: (i, 0)),
            pl.BlockSpec((_EMB_DIM, bb), lambda i: (0, i)),
            pl.BlockSpec((bb, 1), lambda i: (i, 0)),
            pl.BlockSpec((_D_H, _D_IN), lambda i: (0, 0)),
            pl.BlockSpec((1, _D_H), lambda i: (0, 0)),
            pl.BlockSpec((_D_H, 2), lambda i: (0, 0)),
            pl.BlockSpec((2, 1), lambda i: (0, 0)),
        ],
        out_specs=[
            pl.BlockSpec((bb,), lambda i: (i,)),
            pl.BlockSpec((bb,), lambda i: (i,)),
        ],
        out_shape=[
            jax.ShapeDtypeStruct((_B,), jnp.float32),
            jax.ShapeDtypeStruct((_B,), jnp.float32),
        ],
        compiler_params=pltpu.CompilerParams(
            fuse_transposed_lhs_in_matmul=True,
            vmem_limit_bytes=100 * 1024 * 1024),
    )(cls_embeddings, user_matrix_t, is_word_indices, W1.T,
      b1.reshape(1, _D_H), W2, b2.reshape(2, 1))


def kernel(cls_embeddings, user_indices, is_word_indices, user_emb_table,
           W1, b1, W2, b2):
    user_matrix_t = _gather_rows(user_indices, user_emb_table)
    arousal, valence = _mlp(cls_embeddings, user_matrix_t, is_word_indices,
                            W1, b1, W2, b2)
    return (arousal, valence)


# speedup vs baseline: 1.1102x; 1.0001x over previous
import jax
import jax.numpy as jnp
from jax import lax
from jax.experimental import pallas as pl
from jax.experimental.pallas import tpu as pltpu
from jax.experimental.pallas import tpu_sc as plsc

_B = 16384
_ROBERTA_DIM = 768
_EMB_DIM = 4
_D_IN = _ROBERTA_DIM + 1 + _EMB_DIM
_D_H = _D_IN // 2

_NC = 2
_NS = 16
_NW = _NC * _NS
_CHUNK = 128

_ELEMS_PER_W = _EMB_DIM * _B // _NW
_ECHUNKS_PER_W = _ELEMS_PER_W // _CHUNK


def _sc_gather(fidx_hbm, tabf_hbm, out_hbm, fidx_v, vals_v, sem):
    wid = lax.axis_index("s") * _NC + lax.axis_index("c")
    pltpu.sync_copy(fidx_hbm.at[pl.ds(wid * _ECHUNKS_PER_W, _ECHUNKS_PER_W)],
                    fidx_v)
    copies = []
    for j in range(_ECHUNKS_PER_W):
        copies.append(
            pltpu.async_copy(
                tabf_hbm.at[fidx_v.at[j]],
                vals_v.at[pl.ds(j * _CHUNK, _CHUNK)],
                sem,
            )
        )
    for c in copies:
        c.wait()
    pltpu.sync_copy(vals_v, out_hbm.at[pl.ds(wid * _ELEMS_PER_W,
                                             _ELEMS_PER_W)])


@jax.jit
def _gather_rows(user_indices, user_emb_table):
    fidx = (jnp.arange(4, dtype=jnp.int32)[:, None] * 1000000
            + user_indices[None, :])
    fidx = fidx.reshape(_EMB_DIM * _B // _CHUNK, _CHUNK)
    tabf = user_emb_table.T.reshape(-1)
    mesh = plsc.VectorSubcoreMesh(core_axis_name="c", subcore_axis_name="s")
    k = pl.kernel(
        _sc_gather,
        out_type=jax.ShapeDtypeStruct((_B * _EMB_DIM,), jnp.float32),
        mesh=mesh,
        scratch_types=[
            pltpu.VMEM((_ECHUNKS_PER_W, _CHUNK), jnp.int32),
            pltpu.VMEM((_ELEMS_PER_W,), jnp.float32),
            pltpu.SemaphoreType.DMA,
        ],
    )
    return k(fidx, tabf).reshape(_EMB_DIM, _B)


def _mlp_body(cls_ref, usr_ref, isw_ref, w1t_ref, b1_ref,
              w2_ref, b2_ref, aro_ref, val_ref):
    x = cls_ref[...]
    w1t = w1t_ref[...]
    acc = lax.dot_general(x, w1t[:, :_ROBERTA_DIM],
                          (((1,), (1,)), ((), ())),
                          preferred_element_type=jnp.float32)
    acc += lax.dot_general(
        jnp.transpose(usr_ref[...]),
        w1t[:, _ROBERTA_DIM:_ROBERTA_DIM + _EMB_DIM],
        (((1,), (1,)), ((), ())),
        preferred_element_type=jnp.float32)
    acc += isw_ref[...] * jnp.transpose(w1t[:, _D_IN - 1:])
    acc += b1_ref[...]
    h = 0.5 * acc * (1.0 + lax.erf(acc * 0.7071067811865476))
    logits = jnp.dot(h, w2_ref[...], preferred_element_type=jnp.float32)
    logits_t = jnp.transpose(logits) + b2_ref[...]
    aro_ref[...] = jax.nn.sigmoid(logits_t[1])
    val_ref[...] = jax.nn.sigmoid(logits_t[0])


@jax.jit
def _mlp(cls_embeddings, user_matrix_t, is_word_indices, W1, b1, W2, b2):
    bb = 2048
    grid = (_B // bb,)
    return pl.pallas_call(
        _mlp_body,
        grid=grid,
        in_specs=[
            pl.BlockSpec((bb, _ROBERTA_DIM), lambda i: (i, 0)),
            pl.BlockSpec((_EMB_DIM, bb), lambda i: (0, i)),
            pl.BlockSpec((bb, 1), lambda i: (i, 0)),
            pl.BlockSpec((_D_H, _D_IN), lambda i: (0, 0)),
            pl.BlockSpec((1, _D_H), lambda i: (0, 0)),
            pl.BlockSpec((_D_H, 2), lambda i: (0, 0)),
            pl.BlockSpec((2, 1), lambda i: (0, 0)),
        ],
        out_specs=[
            pl.BlockSpec((bb,), lambda i: (i,)),
            pl.BlockSpec((bb,), lambda i: (i,)),
        ],
        out_shape=[
            jax.ShapeDtypeStruct((_B,), jnp.float32),
            jax.ShapeDtypeStruct((_B,), jnp.float32),
        ],
        compiler_params=pltpu.CompilerParams(
            fuse_transposed_lhs_in_matmul=True,
            vmem_limit_bytes=100 * 1024 * 1024),
    )(cls_embeddings, user_matrix_t, is_word_indices, W1.T,
      b1.reshape(1, _D_H), W2, b2.reshape(2, 1))


def kernel(cls_embeddings, user_indices, is_word_indices, user_emb_table,
           W1, b1, W2, b2):
    user_matrix_t = _gather_rows(user_indices, user_emb_table)
    arousal, valence = _mlp(cls_embeddings, user_matrix_t, is_word_indices,
                            W1, b1, W2, b2)
    return (arousal, valence)
